# baseline (speedup 1.0000x reference)
import functools

import jax
import jax.numpy as jnp
from jax import lax
from jax.experimental import pallas as pl
from jax.experimental.pallas import tpu as pltpu
from jax.experimental.pallas import tpu_sc as plsc

EMBED_DIM = 64
CHUNK = 512


@functools.lru_cache(maxsize=None)
def _build(B: int):
    info = plsc.get_sparse_core_info()
    NC, NS = info.num_cores, info.num_subcores
    NW = NC * NS
    assert B % (NW * CHUNK) == 0
    b_per_w = B // NW
    nsteps = b_per_w // CHUNK
    mesh = plsc.VectorSubcoreMesh(core_axis_name="c", subcore_axis_name="s")

    @functools.partial(
        pl.kernel,
        mesh=mesh,
        out_type=jax.ShapeDtypeStruct((B, EMBED_DIM), jnp.float32),
        scratch_types=[
            pltpu.VMEM((CHUNK,), jnp.int32),
            pltpu.VMEM((CHUNK, EMBED_DIM), jnp.float32),
            pltpu.SemaphoreType.DMA,
        ],
        compiler_params=pltpu.CompilerParams(use_tc_tiling_on_sc=False),
    )
    def gather_kernel(idx_hbm, table_hbm, out_hbm, idx_v, rows_v, sem):
        wid = lax.axis_index("s") * NC + lax.axis_index("c")
        base = wid * b_per_w

        def body(s, carry):
            off = base + s * CHUNK
            pltpu.sync_copy(idx_hbm.at[pl.ds(off, CHUNK)], idx_v)
            pltpu.async_copy(table_hbm.at[idx_v], rows_v, sem).wait()
            pltpu.sync_copy(rows_v, out_hbm.at[pl.ds(off, CHUNK)])
            return carry

        lax.fori_loop(0, nsteps, body, 0)

    return gather_kernel


def kernel(input, hidden, table):
    B = input.shape[0] * input.shape[1]
    idx = input.reshape(B).astype(jnp.int32)
    out = _build(B)(idx, table)
    return out.reshape(input.shape[0], input.shape[1], EMBED_DIM)


# speedup vs baseline: 1.0428x; 1.0428x over previous
import functools

import jax
import jax.numpy as jnp
from jax import lax
from jax.experimental import pallas as pl
from jax.experimental.pallas import tpu as pltpu
from jax.experimental.pallas import tpu_sc as plsc

EMBED_DIM = 64
CHUNK = 320
NBUF = 4


@functools.lru_cache(maxsize=None)
def _build(B: int):
    info = plsc.get_sparse_core_info()
    NC, NS = info.num_cores, info.num_subcores
    NW = NC * NS
    b_per_w = B // NW
    nsteps = b_per_w // CHUNK
    assert B % NW == 0 and b_per_w % CHUNK == 0 and nsteps % NBUF == 0
    mesh = plsc.VectorSubcoreMesh(core_axis_name="c", subcore_axis_name="s")

    scratch = [pltpu.VMEM((b_per_w,), jnp.int32)]
    scratch += [pltpu.VMEM((CHUNK, EMBED_DIM), jnp.float32) for _ in range(NBUF)]
    scratch += [pltpu.SemaphoreType.DMA for _ in range(2 * NBUF)]

    @functools.partial(
        pl.kernel,
        mesh=mesh,
        out_type=jax.ShapeDtypeStruct((B, EMBED_DIM), jnp.float32),
        scratch_types=scratch,
        compiler_params=pltpu.CompilerParams(use_tc_tiling_on_sc=False),
    )
    def gather_kernel(idx_hbm, table_hbm, out_hbm, idx_v, *rest):
        rows = rest[:NBUF]
        sem_g = rest[NBUF:2 * NBUF]
        sem_w = rest[2 * NBUF:]
        wid = lax.axis_index("s") * NC + lax.axis_index("c")
        base = wid * b_per_w

        pltpu.sync_copy(idx_hbm.at[pl.ds(base, b_per_w)], idx_v)

        def start_g(s, b):
            pltpu.async_copy(
                table_hbm.at[idx_v.at[pl.ds(s * CHUNK, CHUNK)]], rows[b], sem_g[b])

        def wait_g(b):
            pltpu.make_async_copy(
                table_hbm.at[idx_v.at[pl.ds(0, CHUNK)]], rows[b], sem_g[b]).wait()

        def start_w(s, b):
            pltpu.async_copy(
                rows[b], out_hbm.at[pl.ds(base + s * CHUNK, CHUNK)], sem_w[b])

        def wait_w(b):
            pltpu.make_async_copy(
                rows[b], out_hbm.at[pl.ds(base, CHUNK)], sem_w[b]).wait()

        start_g(0, 0)
        start_g(1, 1)

        def outer(t, carry):
            for b in range(NBUF):
                s = t * NBUF + b
                wait_g(b)
                start_w(s, b)
                b2 = (b + 2) % NBUF

                @pl.when(s + 2 < nsteps)
                def _issue():
                    @pl.when(s >= 2)
                    def _drain():
                        wait_w(b2)
                    start_g(s + 2, b2)
            return carry

        lax.fori_loop(0, nsteps // NBUF, outer, 0)
        for b in range(NBUF):
            wait_w(b)

    return gather_kernel


def kernel(input, hidden, table):
    B = input.shape[0] * input.shape[1]
    idx = input.reshape(B).astype(jnp.int32)
    out = _build(B)(idx, table)
    return out.reshape(input.shape[0], input.shape[1], EMBED_DIM)


# speedup vs baseline: 1.0737x; 1.0297x over previous
import functools

import jax
import jax.numpy as jnp
from jax import lax
from jax.experimental import pallas as pl
from jax.experimental.pallas import tpu as pltpu
from jax.experimental.pallas import tpu_sc as plsc

EMBED_DIM = 64
CHUNK = 320
NBUF = 4


@functools.lru_cache(maxsize=None)
def _build(B: int):
    info = plsc.get_sparse_core_info()
    NC, NS = info.num_cores, info.num_subcores
    NW = NC * NS
    b_per_w = B // NW
    nsteps = b_per_w // CHUNK
    assert B % NW == 0 and b_per_w % CHUNK == 0 and nsteps % NBUF == 0
    mesh = plsc.VectorSubcoreMesh(core_axis_name="c", subcore_axis_name="s")

    scratch = [pltpu.VMEM((b_per_w,), jnp.int32)]
    scratch += [pltpu.VMEM((CHUNK, EMBED_DIM), jnp.float32) for _ in range(NBUF)]
    scratch += [pltpu.SemaphoreType.DMA for _ in range(2 * NBUF)]

    @functools.partial(
        pl.kernel,
        mesh=mesh,
        out_type=jax.ShapeDtypeStruct((B, EMBED_DIM), jnp.float32),
        scratch_types=scratch,
        compiler_params=pltpu.CompilerParams(use_tc_tiling_on_sc=False),
    )
    def gather_kernel(idx_hbm, table_hbm, out_hbm, idx_v, *rest):
        rows = rest[:NBUF]
        sem_g = rest[NBUF:2 * NBUF]
        sem_w = rest[2 * NBUF:]
        wid = lax.axis_index("s") * NC + lax.axis_index("c")
        base = wid * b_per_w

        pltpu.sync_copy(idx_hbm.at[pl.ds(base, b_per_w)], idx_v)

        def start_g(s, b):
            pltpu.async_copy(
                table_hbm.at[idx_v.at[pl.ds(s * CHUNK, CHUNK)]], rows[b], sem_g[b])

        def wait_g(b):
            pltpu.make_async_copy(
                table_hbm.at[idx_v.at[pl.ds(0, CHUNK)]], rows[b], sem_g[b]).wait()

        def start_w(s, b):
            pltpu.async_copy(
                rows[b], out_hbm.at[pl.ds(base + s * CHUNK, CHUNK)], sem_w[b])

        def wait_w(b):
            pltpu.make_async_copy(
                rows[b], out_hbm.at[pl.ds(base, CHUNK)], sem_w[b]).wait()

        start_g(0, 0)
        start_g(1, 1)

        def outer(t, carry):
            for b in range(NBUF):
                s = t * NBUF + b
                wait_g(b)
                start_w(s, b)
                b2 = (b + 2) % NBUF

                @pl.when(s + 2 < nsteps)
                def _issue():
                    @pl.when(s >= 2)
                    def _drain():
                        wait_w(b2)
                    start_g(s + 2, b2)
            return carry

        lax.fori_loop(0, nsteps // NBUF, outer, 0)
        for b in range(NBUF):
            wait_w(b)

    return gather_kernel


def kernel(input, hidden, table):
    BATCH, HIST = input.shape
    B = BATCH * HIST
    idx = input.T.reshape(B).astype(jnp.int32)
    out = _build(B)(idx, table)
    return out.reshape(HIST, BATCH, EMBED_DIM).transpose(1, 0, 2)
